# baseline (speedup 1.0000x reference)
import functools

import jax
import jax.numpy as jnp
from jax import lax
from jax.experimental import pallas as pl
from jax.experimental.pallas import tpu as pltpu
from jax.experimental.pallas import tpu_sc as plsc

B = 1024
L = 200
V = 100000

NS = 4
BS = B // NS
BB = 16
NW = 32
EPG = 8 * L
NVEC = EPG // 16
CW = 4992
NCH = 20
TW = 160
TS = NCH * CW
TK = NCH


def _val_idx_block(loc_ref, m_ref, rw_ref, fw_ref, row_ref, chk_ref, col_ref,
                   val_ref):
    loc = loc_ref[...]
    m = m_ref[...]
    rw = rw_ref[0]
    fw = fw_ref[0]
    t = lax.broadcasted_iota(jnp.int32, (1, L), 1).astype(jnp.float32)
    rf = jnp.exp((jnp.float32(L - 1) - t) * jnp.log(rw))
    rv = rf * m
    eq = (loc[:, :, None] == loc[:, None, :]).astype(jnp.float32)
    cnt = jnp.sum(eq * m[:, None, :], axis=2)
    rec = jnp.max(eq * rv[:, None, :], axis=2)
    maxf = jnp.maximum(jnp.max(cnt, axis=1, keepdims=True), 1.0)
    val_ref[...] = rec + fw * cnt / maxf
    i = pl.program_id(0)
    rows = i * BB + lax.broadcasted_iota(jnp.int32, (BB, L), 0)
    row_ref[...] = rows & 7
    k = loc // CW
    chk_ref[...] = k
    col_ref[...] = loc - k * CW


def _val_idx_call(loc_seq, mask, rw, fw):
    return pl.pallas_call(
        _val_idx_block,
        grid=(BS // BB,),
        in_specs=[
            pl.BlockSpec((BB, L), lambda i: (i, 0)),
            pl.BlockSpec((BB, L), lambda i: (i, 0)),
            pl.BlockSpec(memory_space=pltpu.SMEM),
            pl.BlockSpec(memory_space=pltpu.SMEM),
        ],
        out_specs=[
            pl.BlockSpec((BB, L), lambda i: (i, 0)),
            pl.BlockSpec((BB, L), lambda i: (i, 0)),
            pl.BlockSpec((BB, L), lambda i: (i, 0)),
            pl.BlockSpec((BB, L), lambda i: (i, 0)),
        ],
        out_shape=[
            jax.ShapeDtypeStruct((BS, L), jnp.int32),
            jax.ShapeDtypeStruct((BS, L), jnp.int32),
            jax.ShapeDtypeStruct((BS, L), jnp.int32),
            jax.ShapeDtypeStruct((BS, L), jnp.float32),
        ],
    )(loc_seq, mask, rw, fw)


def _sc_body(row_hbm, chk_hbm, col_hbm, val_hbm, out_hbm,
             row_v, chk_v, col_v, val_v, buf0, buf1, tbuf, sem0, sem1):
    c = lax.axis_index("c")
    s = lax.axis_index("s")
    w = c * 16 + s

    z16f = jnp.zeros((16,), jnp.float32)

    def zmain(i, carry):
        r = i // (CW // 16)
        o = (i % (CW // 16)) * 16
        buf0[r, pl.ds(o, 16)] = z16f
        buf1[r, pl.ds(o, 16)] = z16f
        return carry

    lax.fori_loop(0, 8 * (CW // 16), zmain, 0)

    def ztail(i, carry):
        r = i // (TW // 16)
        o = (i % (TW // 16)) * 16
        tbuf[r, pl.ds(o, 16)] = z16f
        return carry

    lax.fori_loop(0, 8 * (TW // 16), ztail, 0)

    pltpu.sync_copy(row_hbm.at[w], row_v)
    pltpu.sync_copy(chk_hbm.at[w], chk_v)
    pltpu.sync_copy(col_hbm.at[w], col_v)
    pltpu.sync_copy(val_hbm.at[w], val_v)

    bufs = (buf0, buf1)
    sems = (sem0, sem1)

    def scan(k, buf, zero):
        def body(i, carry):
            rv = row_v[0, pl.ds(i * 16, 16)]
            kv = chk_v[0, pl.ds(i * 16, 16)]
            cv = col_v[0, pl.ds(i * 16, 16)]
            vv = z16f if zero else val_v[0, pl.ds(i * 16, 16)]
            plsc.store_scatter(buf, [rv, cv], vv, mask=kv == k)
            return carry
        lax.fori_loop(0, NVEC, body, 0)

    def chunk_dma(k, buf, sem):
        cs = pl.multiple_of(k * CW, 128)
        return pltpu.make_async_copy(
            buf, out_hbm.at[pl.ds(w * 8, 8), pl.ds(cs, CW)], sem)

    def piped(j, carry):
        for sl in (0, 1):
            k = 2 * j + sl

            @pl.when(j > 0)
            def _():
                chunk_dma(k - 2, bufs[sl], sems[sl]).wait()
                scan(k - 2, bufs[sl], zero=True)

            scan(k, bufs[sl], zero=False)
            chunk_dma(k, bufs[sl], sems[sl]).start()
        return carry

    lax.fori_loop(0, NCH // 2, piped, 0)
    for sl in (0, 1):
        chunk_dma(NCH - 2 + sl, bufs[sl], sems[sl]).wait()

    scan(TK, tbuf, zero=False)
    tdma = pltpu.make_async_copy(
        tbuf, out_hbm.at[pl.ds(w * 8, 8), pl.ds(TS, TW)], sem0)
    tdma.start()
    tdma.wait()


@functools.cache
def _sc_call():
    mesh = plsc.VectorSubcoreMesh(core_axis_name="c", subcore_axis_name="s")
    return pl.kernel(
        _sc_body,
        out_type=jax.ShapeDtypeStruct((BS, V), jnp.float32),
        mesh=mesh,
        compiler_params=pltpu.CompilerParams(needs_layout_passes=False),
        scratch_types=[
            pltpu.VMEM((1, EPG), jnp.int32),
            pltpu.VMEM((1, EPG), jnp.int32),
            pltpu.VMEM((1, EPG), jnp.int32),
            pltpu.VMEM((1, EPG), jnp.float32),
            pltpu.VMEM((8, CW), jnp.float32),
            pltpu.VMEM((8, CW), jnp.float32),
            pltpu.VMEM((8, TW), jnp.float32),
            pltpu.SemaphoreType.DMA,
            pltpu.SemaphoreType.DMA,
        ],
    )


def kernel(loc_seq, mask, recency_weight, frequency_weight):
    rw = jnp.asarray(recency_weight, jnp.float32).reshape(1)
    fw = jnp.asarray(frequency_weight, jnp.float32).reshape(1)
    outs = []
    for i in range(NS):
        sl = slice(i * BS, (i + 1) * BS)
        row, chk, col, val = _val_idx_call(loc_seq[sl], mask[sl], rw, fw)
        outs.append(_sc_call()(
            row.reshape(NW, 1, EPG),
            chk.reshape(NW, 1, EPG),
            col.reshape(NW, 1, EPG),
            val.reshape(NW, 1, EPG),
        ))
    acc = lax.empty((B, V), jnp.float32)
    for i, o in enumerate(outs):
        acc = lax.dynamic_update_slice(acc, o, (i * BS, 0))
    return acc


# speedup vs baseline: 1.2429x; 1.2429x over previous
import functools

import jax
import jax.numpy as jnp
from jax import lax
from jax.experimental import pallas as pl
from jax.experimental.pallas import tpu as pltpu
from jax.experimental.pallas import tpu_sc as plsc

B = 1024
L = 200
V = 100000

BB = 16
NW = 32
GPT = 4
EPG = 8 * L
NVEC = EPG // 16
CW = 4992
NCH = 20
TW = 160
TS = NCH * CW
TK = NCH


def _val_idx_block(loc_ref, m_ref, rw_ref, fw_ref, row_ref, chk_ref, col_ref,
                   val_ref):
    loc = loc_ref[...]
    m = m_ref[...]
    rw = rw_ref[0]
    fw = fw_ref[0]
    t = lax.broadcasted_iota(jnp.int32, (1, L), 1).astype(jnp.float32)
    rf = jnp.exp((jnp.float32(L - 1) - t) * jnp.log(rw))
    rv = rf * m
    eq = (loc[:, :, None] == loc[:, None, :]).astype(jnp.float32)
    cnt = jnp.sum(eq * m[:, None, :], axis=2)
    rec = jnp.max(eq * rv[:, None, :], axis=2)
    maxf = jnp.maximum(jnp.max(cnt, axis=1, keepdims=True), 1.0)
    val_ref[...] = rec + fw * cnt / maxf
    i = pl.program_id(0)
    rows = i * BB + lax.broadcasted_iota(jnp.int32, (BB, L), 0)
    row_ref[...] = rows & 7
    k = loc // CW
    chk_ref[...] = k
    col_ref[...] = loc - k * CW


def _val_idx_call(loc_seq, mask, rw, fw):
    return pl.pallas_call(
        _val_idx_block,
        grid=(B // BB,),
        in_specs=[
            pl.BlockSpec((BB, L), lambda i: (i, 0)),
            pl.BlockSpec((BB, L), lambda i: (i, 0)),
            pl.BlockSpec(memory_space=pltpu.SMEM),
            pl.BlockSpec(memory_space=pltpu.SMEM),
        ],
        out_specs=[
            pl.BlockSpec((BB, L), lambda i: (i, 0)),
            pl.BlockSpec((BB, L), lambda i: (i, 0)),
            pl.BlockSpec((BB, L), lambda i: (i, 0)),
            pl.BlockSpec((BB, L), lambda i: (i, 0)),
        ],
        out_shape=[
            jax.ShapeDtypeStruct((B, L), jnp.int32),
            jax.ShapeDtypeStruct((B, L), jnp.int32),
            jax.ShapeDtypeStruct((B, L), jnp.int32),
            jax.ShapeDtypeStruct((B, L), jnp.float32),
        ],
    )(loc_seq, mask, rw, fw)


def _sc_body(out_hbm, row_hbm, chk_hbm, col_hbm, val_hbm,
             row_v, chk_v, col_v, val_v, buf0, buf1, tbuf, sem0, sem1):
    c = lax.axis_index("c")
    s = lax.axis_index("s")
    w = c * 16 + s

    z16f = jnp.zeros((16,), jnp.float32)

    def zmain(i, carry):
        r = i // (CW // 16)
        o = (i % (CW // 16)) * 16
        buf0[r, pl.ds(o, 16)] = z16f
        buf1[r, pl.ds(o, 16)] = z16f
        return carry

    lax.fori_loop(0, 8 * (CW // 16), zmain, 0)

    def ztail(i, carry):
        r = i // (TW // 16)
        o = (i % (TW // 16)) * 16
        tbuf[r, pl.ds(o, 16)] = z16f
        return carry

    lax.fori_loop(0, 8 * (TW // 16), ztail, 0)

    pltpu.sync_copy(row_hbm.at[w], row_v)
    pltpu.sync_copy(chk_hbm.at[w], chk_v)
    pltpu.sync_copy(col_hbm.at[w], col_v)
    pltpu.sync_copy(val_hbm.at[w], val_v)

    bufs = (buf0, buf1)
    sems = (sem0, sem1)

    def scan(a, k, buf, zero):
        def body(i, carry):
            rv = row_v[a, pl.ds(i * 16, 16)]
            kv = chk_v[a, pl.ds(i * 16, 16)]
            cv = col_v[a, pl.ds(i * 16, 16)]
            vv = z16f if zero else val_v[a, pl.ds(i * 16, 16)]
            plsc.store_scatter(buf, [rv, cv], vv, mask=kv == k)
            return carry
        lax.fori_loop(0, NVEC, body, 0)

    def chunk_dma(ch, buf, sem):
        a = ch // NCH
        k = ch % NCH
        g = (w * GPT + a) * 8
        cs = pl.multiple_of(k * CW, 128)
        return pltpu.make_async_copy(
            buf, out_hbm.at[pl.ds(g, 8), pl.ds(cs, CW)], sem)

    def piped(j, carry):
        for sl in (0, 1):
            ch = 2 * j + sl

            @pl.when(j > 0)
            def _():
                prev = ch - 2
                chunk_dma(prev, bufs[sl], sems[sl]).wait()
                scan(prev // NCH, prev % NCH, bufs[sl], zero=True)

            scan(ch // NCH, ch % NCH, bufs[sl], zero=False)
            chunk_dma(ch, bufs[sl], sems[sl]).start()
        return carry

    lax.fori_loop(0, (GPT * NCH) // 2, piped, 0)
    for sl in (0, 1):
        chunk_dma(GPT * NCH - 2 + sl, bufs[sl], sems[sl]).wait()

    def tail(a, carry):
        g = (w * GPT + a) * 8
        scan(a, TK, tbuf, zero=False)
        tdma = pltpu.make_async_copy(
            tbuf, out_hbm.at[pl.ds(g, 8), pl.ds(TS, TW)], sem0)
        tdma.start()
        tdma.wait()
        scan(a, TK, tbuf, zero=True)
        return carry

    lax.fori_loop(0, GPT, tail, 0)


@functools.cache
def _sc_call():
    mesh = plsc.VectorSubcoreMesh(core_axis_name="c", subcore_axis_name="s")
    return pl.kernel(
        _sc_body,
        out_type=(),
        mesh=mesh,
        compiler_params=pltpu.CompilerParams(needs_layout_passes=False),
        scratch_types=[
            pltpu.VMEM((GPT, EPG), jnp.int32),
            pltpu.VMEM((GPT, EPG), jnp.int32),
            pltpu.VMEM((GPT, EPG), jnp.int32),
            pltpu.VMEM((GPT, EPG), jnp.float32),
            pltpu.VMEM((8, CW), jnp.float32),
            pltpu.VMEM((8, CW), jnp.float32),
            pltpu.VMEM((8, TW), jnp.float32),
            pltpu.SemaphoreType.DMA,
            pltpu.SemaphoreType.DMA,
        ],
    )


def kernel(loc_seq, mask, recency_weight, frequency_weight):
    rw = jnp.asarray(recency_weight, jnp.float32).reshape(1)
    fw = jnp.asarray(frequency_weight, jnp.float32).reshape(1)
    row, chk, col, val = _val_idx_call(loc_seq, mask, rw, fw)
    out_ref = jax.new_ref(lax.empty((B, V), jnp.float32))
    _sc_call()(out_ref,
               row.reshape(NW, GPT, EPG),
               chk.reshape(NW, GPT, EPG),
               col.reshape(NW, GPT, EPG),
               val.reshape(NW, GPT, EPG))
    return jax.freeze(out_ref)


# speedup vs baseline: 1.3262x; 1.0670x over previous
import functools

import jax
import jax.numpy as jnp
from jax import lax
from jax.experimental import pallas as pl
from jax.experimental.pallas import tpu as pltpu
from jax.experimental.pallas import tpu_sc as plsc

B = 1024
L = 200
V = 100000

NH = 2
HB = B // NH
BB = 16
NW = 32
GPT = HB // (8 * NW)
EPG = 8 * L
NVEC = EPG // 16
CW = 4992
NCH = 20
TW = 160
TS = NCH * CW
TK = NCH


def _val_idx_block(loc_ref, m_ref, rw_ref, fw_ref, row_ref, chk_ref, col_ref,
                   val_ref):
    loc = loc_ref[...]
    m = m_ref[...]
    rw = rw_ref[0]
    fw = fw_ref[0]
    t = lax.broadcasted_iota(jnp.int32, (1, L), 1).astype(jnp.float32)
    rf = jnp.exp((jnp.float32(L - 1) - t) * jnp.log(rw))
    rv = rf * m
    eq = (loc[:, :, None] == loc[:, None, :]).astype(jnp.float32)
    cnt = jnp.sum(eq * m[:, None, :], axis=2)
    rec = jnp.max(eq * rv[:, None, :], axis=2)
    maxf = jnp.maximum(jnp.max(cnt, axis=1, keepdims=True), 1.0)
    val_ref[...] = rec + fw * cnt / maxf
    i = pl.program_id(0)
    rows = i * BB + lax.broadcasted_iota(jnp.int32, (BB, L), 0)
    row_ref[...] = rows & 7
    k = loc // CW
    chk_ref[...] = k
    col_ref[...] = loc - k * CW


def _val_idx_call(loc_seq, mask, rw, fw):
    return pl.pallas_call(
        _val_idx_block,
        grid=(HB // BB,),
        in_specs=[
            pl.BlockSpec((BB, L), lambda i: (i, 0)),
            pl.BlockSpec((BB, L), lambda i: (i, 0)),
            pl.BlockSpec(memory_space=pltpu.SMEM),
            pl.BlockSpec(memory_space=pltpu.SMEM),
        ],
        out_specs=[
            pl.BlockSpec((BB, L), lambda i: (i, 0)),
            pl.BlockSpec((BB, L), lambda i: (i, 0)),
            pl.BlockSpec((BB, L), lambda i: (i, 0)),
            pl.BlockSpec((BB, L), lambda i: (i, 0)),
        ],
        out_shape=[
            jax.ShapeDtypeStruct((HB, L), jnp.int32),
            jax.ShapeDtypeStruct((HB, L), jnp.int32),
            jax.ShapeDtypeStruct((HB, L), jnp.int32),
            jax.ShapeDtypeStruct((HB, L), jnp.float32),
        ],
    )(loc_seq, mask, rw, fw)


def _make_sc_body(goff):
    def _sc_body(out_hbm, row_hbm, chk_hbm, col_hbm, val_hbm,
                 row_v, chk_v, col_v, val_v, buf0, buf1, tbuf, sem0, sem1):
        c = lax.axis_index("c")
        s = lax.axis_index("s")
        w = c * 16 + s

        z16f = jnp.zeros((16,), jnp.float32)

        def zmain(i, carry):
            r = i // (CW // 16)
            o = (i % (CW // 16)) * 16
            buf0[r, pl.ds(o, 16)] = z16f
            buf1[r, pl.ds(o, 16)] = z16f
            return carry

        lax.fori_loop(0, 8 * (CW // 16), zmain, 0)

        def ztail(i, carry):
            r = i // (TW // 16)
            o = (i % (TW // 16)) * 16
            tbuf[r, pl.ds(o, 16)] = z16f
            return carry

        lax.fori_loop(0, 8 * (TW // 16), ztail, 0)

        pltpu.sync_copy(row_hbm.at[w], row_v)
        pltpu.sync_copy(chk_hbm.at[w], chk_v)
        pltpu.sync_copy(col_hbm.at[w], col_v)
        pltpu.sync_copy(val_hbm.at[w], val_v)

        bufs = (buf0, buf1)
        sems = (sem0, sem1)

        def scan(a, k, buf, zero):
            def body(i, carry):
                rv = row_v[a, pl.ds(i * 16, 16)]
                kv = chk_v[a, pl.ds(i * 16, 16)]
                cv = col_v[a, pl.ds(i * 16, 16)]
                vv = z16f if zero else val_v[a, pl.ds(i * 16, 16)]
                plsc.store_scatter(buf, [rv, cv], vv, mask=kv == k)
                return carry
            lax.fori_loop(0, NVEC, body, 0)

        def chunk_dma(ch, buf, sem):
            a = ch // NCH
            k = ch % NCH
            g = (goff + w * GPT + a) * 8
            cs = pl.multiple_of(k * CW, 128)
            return pltpu.make_async_copy(
                buf, out_hbm.at[pl.ds(g, 8), pl.ds(cs, CW)], sem)

        def piped(j, carry):
            for sl in (0, 1):
                ch = 2 * j + sl

                @pl.when(j > 0)
                def _():
                    prev = ch - 2
                    chunk_dma(prev, bufs[sl], sems[sl]).wait()
                    scan(prev // NCH, prev % NCH, bufs[sl], zero=True)

                scan(ch // NCH, ch % NCH, bufs[sl], zero=False)
                chunk_dma(ch, bufs[sl], sems[sl]).start()
            return carry

        lax.fori_loop(0, (GPT * NCH) // 2, piped, 0)
        for sl in (0, 1):
            chunk_dma(GPT * NCH - 2 + sl, bufs[sl], sems[sl]).wait()

        def tail(a, carry):
            g = (goff + w * GPT + a) * 8
            scan(a, TK, tbuf, zero=False)
            tdma = pltpu.make_async_copy(
                tbuf, out_hbm.at[pl.ds(g, 8), pl.ds(TS, TW)], sem0)
            tdma.start()
            tdma.wait()
            scan(a, TK, tbuf, zero=True)
            return carry

        lax.fori_loop(0, GPT, tail, 0)

    return _sc_body


@functools.cache
def _sc_call(half):
    mesh = plsc.VectorSubcoreMesh(core_axis_name="c", subcore_axis_name="s")
    return pl.kernel(
        _make_sc_body(half * NW * GPT),
        out_type=(),
        mesh=mesh,
        compiler_params=pltpu.CompilerParams(needs_layout_passes=False),
        scratch_types=[
            pltpu.VMEM((GPT, EPG), jnp.int32),
            pltpu.VMEM((GPT, EPG), jnp.int32),
            pltpu.VMEM((GPT, EPG), jnp.int32),
            pltpu.VMEM((GPT, EPG), jnp.float32),
            pltpu.VMEM((8, CW), jnp.float32),
            pltpu.VMEM((8, CW), jnp.float32),
            pltpu.VMEM((8, TW), jnp.float32),
            pltpu.SemaphoreType.DMA,
            pltpu.SemaphoreType.DMA,
        ],
    )


def kernel(loc_seq, mask, recency_weight, frequency_weight):
    rw = jnp.asarray(recency_weight, jnp.float32).reshape(1)
    fw = jnp.asarray(frequency_weight, jnp.float32).reshape(1)
    out_ref = jax.new_ref(lax.empty((B, V), jnp.float32))
    for h in range(NH):
        sl = slice(h * HB, (h + 1) * HB)
        row, chk, col, val = _val_idx_call(loc_seq[sl], mask[sl], rw, fw)
        _sc_call(h)(out_ref,
                    row.reshape(NW, GPT, EPG),
                    chk.reshape(NW, GPT, EPG),
                    col.reshape(NW, GPT, EPG),
                    val.reshape(NW, GPT, EPG))
    return jax.freeze(out_ref)
